# baseline (speedup 1.0000x reference)
import jax
import jax.numpy as jnp
from jax.experimental import pallas as pl


_BLK = 800


def _body(x_ref, o0_ref, o1_ref):
    x = x_ref[...]
    o0_ref[...] = x[:, 0:160]
    o1_ref[...] = x[:, 96:256]


def kernel(X):
    M, N = X.shape
    grid = (M // _BLK,)
    p0, p1 = pl.pallas_call(
        _body,
        grid=grid,
        in_specs=[pl.BlockSpec((_BLK, N), lambda i: (i, 0))],
        out_specs=[
            pl.BlockSpec((_BLK, 160), lambda i: (i, 0)),
            pl.BlockSpec((_BLK, 160), lambda i: (i, 0)),
        ],
        out_shape=[
            jax.ShapeDtypeStruct((M, 160), X.dtype),
            jax.ShapeDtypeStruct((M, 160), X.dtype),
        ],
    )(X)
    return (p0, p1)
